# baseline (speedup 1.0000x reference)
import functools

import jax
import jax.numpy as jnp
from jax import lax
from jax.experimental import pallas as pl
from jax.experimental.pallas import tpu as pltpu
from jax.experimental.pallas import tpu_sc as plsc

NQ = 8
V = 100000
D = 32
B = 4096
T = 50
W = NQ * T

NBW = 128
GL = 128
GPC = NBW * NQ // GL


VB = 160
BPT = V // VB
BPW = 640 // 32


def _transpose_call():
    info_nc, info_ns = 2, 16
    try:
        info = plsc.get_sparse_core_info()
        info_nc, info_ns = info.num_cores, info.num_subcores
    except Exception:
        pass

    mesh = plsc.VectorSubcoreMesh(core_axis_name="c", subcore_axis_name="s")

    @functools.partial(
        pl.kernel,
        mesh=mesh,
        out_type=jax.ShapeDtypeStruct((NQ * V, D), jnp.float32),
        compiler_params=pltpu.CompilerParams(
            use_tc_tiling_on_sc=False, needs_layout_passes=False),
        scratch_types=[
            pltpu.VMEM((1, D, VB), jnp.float32),
            pltpu.VMEM((1, D, VB), jnp.float32),
            pltpu.VMEM((VB, 33), jnp.float32),
            pltpu.VMEM((VB, 33), jnp.float32),
            pltpu.SemaphoreType.DMA,
            pltpu.SemaphoreType.DMA,
            pltpu.SemaphoreType.DMA,
            pltpu.SemaphoreType.DMA,
        ],
    )
    def tk(tt_hbm, out_hbm, in0, in1, outb0, outb1,
           isem0, isem1, osem0, osem1):
        wid = lax.axis_index("s") * info_nc + lax.axis_index("c")
        ins = (in0, in1)
        outs = (outb0, outb1)
        isems = (isem0, isem1)
        osems = (osem0, osem1)
        lane = lax.iota(jnp.int32, 16)
        zero16 = lane * 0

        def v0_of(i):
            return jnp.minimum((wid * BPW + i) * VB, V - VB)

        def start_in(q, i, buf):
            pltpu.async_copy(
                tt_hbm.at[pl.ds(q, 1), :, pl.ds(v0_of(i), VB)],
                ins[buf], isems[buf],
            )

        def wait_in(q, buf):
            pltpu.make_async_copy(
                tt_hbm.at[pl.ds(q, 1), :, pl.ds(0, VB)],
                ins[buf], isems[buf],
            ).wait()

        def compute(buf):
            iref = ins[buf]
            oref = outs[buf]

            def e_body(e, carry):
                col = zero16 + e
                for kk in range(VB // 16):
                    vvec = iref[0, e, pl.ds(kk * 16, 16)]
                    plsc.store_scatter(oref, [lane + (kk * 16), col], vvec)
                return carry

            lax.fori_loop(0, D, e_body, 0)

        def out_dst(q, i):
            return out_hbm.at[pl.ds(q * V + v0_of(i), VB)]

        def start_out(q, i, buf):
            pltpu.async_copy(
                outs[buf].at[:, pl.ds(0, D)], out_dst(q, i), osems[buf])

        def wait_out(q, i, buf):
            pltpu.make_async_copy(
                outs[buf].at[:, pl.ds(0, D)], out_dst(q, i), osems[buf]
            ).wait()

        for q in range(NQ):
            start_in(q, 0, 0)

            def stage(i, buf, q=q):
                @pl.when(i + 1 < BPW)
                def _():
                    start_in(q, i + 1, (buf + 1) % 2)

                wait_in(q, buf)

                @pl.when(i >= 2)
                def _():
                    wait_out(q, i - 2, buf)

                compute(buf)
                start_out(q, i, buf)

            def outer(g, carry, q=q):
                stage(2 * g, 0)
                stage(2 * g + 1, 1)
                return carry

            lax.fori_loop(0, BPW // 2, outer, 0)

            wait_out(q, BPW - 2, 0)
            wait_out(q, BPW - 1, 1)

    return tk


def _sc_call():
    info_nc, info_ns = 2, 16
    try:
        info = plsc.get_sparse_core_info()
        info_nc, info_ns = info.num_cores, info.num_subcores
    except Exception:
        pass
    NW = info_nc * info_ns
    assert B // NW == NBW

    mesh = plsc.VectorSubcoreMesh(core_axis_name="c", subcore_axis_name="s")

    @functools.partial(
        pl.kernel,
        mesh=mesh,
        out_type=jax.ShapeDtypeStruct((T * D // 8, NW, 8 * NBW), jnp.float32),
        compiler_params=pltpu.CompilerParams(
            use_tc_tiling_on_sc=False, needs_layout_passes=False),
        scratch_types=[
            pltpu.VMEM((NBW * W,), jnp.int32),
            pltpu.VMEM((GPC, GL), jnp.int32),
            pltpu.VMEM((GPC, GL), jnp.int32),
            pltpu.VMEM((NBW * NQ, D), jnp.float32),
            pltpu.VMEM((NBW * NQ, D), jnp.float32),
            pltpu.VMEM((D // 8, 1, 8 * NBW), jnp.float32),
            pltpu.VMEM((D // 8, 1, 8 * NBW), jnp.float32),
            pltpu.VMEM((NQ, 16), jnp.float32),
            pltpu.SemaphoreType.DMA,
            pltpu.SemaphoreType.DMA,
            pltpu.SemaphoreType.DMA,
            pltpu.SemaphoreType.DMA,
            pltpu.SemaphoreType.DMA,
        ],
    )
    def k(tok_hbm, table_hbm, w_hbm, out_hbm,
          tokv, idx0, idx1, rows0, rows1, outb0, outb1, wv,
          tsem, gsem0, gsem1, osem0, osem1):
        wid = lax.axis_index("s") * info_nc + lax.axis_index("c")
        idxs = (idx0, idx1)
        rows = (rows0, rows1)
        outs = (outb0, outb1)
        gsems = (gsem0, gsem1)
        osems = (osem0, osem1)

        pltpu.sync_copy(w_hbm, wv)
        w = [wv[q, :] for q in range(NQ)]
        lane = lax.iota(jnp.int32, 16)
        bstride = lane * W
        scat_q = lane * 0
        d0a = lane // 8
        d0b = d0a + 2
        zero16 = lane * 0
        d2base = (lane % 8) * NBW

        pltpu.async_copy(
            tok_hbm.at[pl.ds(wid * (NBW * W), NBW * W)], tokv, tsem
        ).wait()

        def issue(c, buf):
            for g in range(GPC):
                for q in range(NQ):
                    toks16 = plsc.load_gather(
                        tokv, [bstride + (g * 16 * W + q * T + c)]
                    )
                    plsc.store_scatter(
                        idxs[buf].at[g], [lane * NQ + q], toks16 + (q * V)
                    )
            for g in range(GPC):
                pltpu.async_copy(
                    table_hbm.at[idxs[buf].at[g]],
                    rows[buf].at[pl.ds(g * GL, GL)],
                    gsems[buf],
                )

        def wait_gathers(buf):
            pltpu.make_async_copy(
                table_hbm.at[pl.ds(0, NBW * NQ)], rows[buf], gsems[buf]
            ).wait()

        def out_slice(c):
            return out_hbm.at[pl.ds(c * (D // 8), D // 8), pl.ds(wid, 1)]

        def compute(buf):
            rref = rows[buf]
            oref = outs[buf]

            def row_body(b, carry):
                base = b * NQ
                a0 = w[0] * rref[base, pl.ds(0, 16)]
                a1 = w[0] * rref[base, pl.ds(16, 16)]
                for q in range(1, NQ):
                    a0 = a0 + w[q] * rref[base + q, pl.ds(0, 16)]
                    a1 = a1 + w[q] * rref[base + q, pl.ds(16, 16)]
                d2 = d2base + b
                plsc.store_scatter(oref, [d0a, zero16, d2], a0)
                plsc.store_scatter(oref, [d0b, zero16, d2], a1)
                return carry

            lax.fori_loop(0, NBW, row_body, 0)

        def stage(c, buf):
            @pl.when(c + 1 < T)
            def _():
                issue(c + 1, (buf + 1) % 2)

            wait_gathers(buf)

            @pl.when(c >= 2)
            def _():
                pltpu.make_async_copy(outs[buf], out_slice(c), osems[buf]).wait()

            compute(buf)
            pltpu.async_copy(outs[buf], out_slice(c), osems[buf])

        issue(0, 0)

        def outer(g, carry):
            stage(2 * g, 0)
            stage(2 * g + 1, 1)
            return carry

        lax.fori_loop(0, T // 2, outer, 0)

        pltpu.make_async_copy(outs[0], out_slice(T - 2), osems[0]).wait()
        pltpu.make_async_copy(outs[1], out_slice(T - 1), osems[1]).wait()

    return k


def kernel(tokens, tables, weights):
    flat_table = _transpose_call()(tables.transpose(0, 2, 1))
    w16 = jnp.broadcast_to(weights.astype(jnp.float32)[:, None], (NQ, 16))
    out3 = _sc_call()(tokens.reshape(B * W), flat_table, w16)
    out5 = out3.reshape(T, D // 8, 32, 8, NBW)
    return out5.transpose(2, 4, 0, 1, 3).reshape(B, T, D)


# speedup vs baseline: 1.3937x; 1.3937x over previous
import functools

import jax
import jax.numpy as jnp
from jax import lax
from jax.experimental import pallas as pl
from jax.experimental.pallas import tpu as pltpu
from jax.experimental.pallas import tpu_sc as plsc

NQ = 8
V = 100000
D = 32
B = 4096
T = 50
W = NQ * T

NBW = 128
GL = 128
GPC = NBW * NQ // GL


def _sc_call():
    info_nc, info_ns = 2, 16
    try:
        info = plsc.get_sparse_core_info()
        info_nc, info_ns = info.num_cores, info.num_subcores
    except Exception:
        pass
    NW = info_nc * info_ns
    assert B // NW == NBW

    mesh = plsc.VectorSubcoreMesh(core_axis_name="c", subcore_axis_name="s")

    @functools.partial(
        pl.kernel,
        mesh=mesh,
        out_type=jax.ShapeDtypeStruct((T * D // 8, NW, 8, NBW), jnp.float32),
        compiler_params=pltpu.CompilerParams(
            use_tc_tiling_on_sc=False, needs_layout_passes=False),
        scratch_types=[
            pltpu.VMEM((NBW, W + 1), jnp.int32),
            pltpu.VMEM((GPC, GL), jnp.int32),
            pltpu.VMEM((GPC, GL), jnp.int32),
            pltpu.VMEM((NBW * NQ, D), jnp.float32),
            pltpu.VMEM((NBW * NQ, D), jnp.float32),
            pltpu.VMEM((1, 1, D, NBW + 1), jnp.float32),
            pltpu.VMEM((1, 1, D, NBW + 1), jnp.float32),
            pltpu.VMEM((NQ, 16), jnp.float32),
            pltpu.SemaphoreType.DMA,
            pltpu.SemaphoreType.DMA,
            pltpu.SemaphoreType.DMA,
            pltpu.SemaphoreType.DMA,
            pltpu.SemaphoreType.DMA,
        ],
    )
    def k(tok_hbm, table_hbm, w_hbm, out_hbm,
          tokv, idx0, idx1, rows0, rows1, outb0, outb1, wv,
          tsem, gsem0, gsem1, osem0, osem1):
        wid = lax.axis_index("s") * info_nc + lax.axis_index("c")
        idxs = (idx0, idx1)
        rows = (rows0, rows1)
        outs = (outb0, outb1)
        gsems = (gsem0, gsem1)
        osems = (osem0, osem1)

        pltpu.sync_copy(w_hbm, wv)
        w = [wv[q, :] for q in range(NQ)]
        lane = lax.iota(jnp.int32, 16)
        zero16 = lane * 0

        pltpu.async_copy(
            tok_hbm.at[pl.ds(wid * NBW, NBW)], tokv.at[:, pl.ds(0, W)], tsem
        ).wait()

        def issue(c, buf):
            for g in range(GPC):
                for q in range(NQ):
                    toks16 = plsc.load_gather(
                        tokv, [lane + g * 16, zero16 + (q * T + c)]
                    )
                    idxs[buf][g, pl.ds(q * 16, 16)] = toks16 + (q * V)
            for g in range(GPC):
                pltpu.async_copy(
                    table_hbm.at[idxs[buf].at[g]],
                    rows[buf].at[pl.ds(g * GL, GL)],
                    gsems[buf],
                )

        def wait_gathers(buf):
            pltpu.make_async_copy(
                table_hbm.at[pl.ds(0, NBW * NQ)], rows[buf], gsems[buf]
            ).wait()

        def out_pairs(c, buf):
            return [
                (outs[buf].at[:, :, pl.ds(eb * 8, 8), pl.ds(0, NBW)],
                 out_hbm.at[pl.ds(c * (D // 8) + eb, 1), pl.ds(wid, 1)])
                for eb in range(D // 8)
            ]

        def compute(buf):
            rref = rows[buf]
            oref = outs[buf]

            def row_body(b, carry):
                base = lax.shift_left(lax.shift_right_logical(b, 4), 7) + \
                    lax.bitwise_and(b, 15)
                a0 = w[0] * rref[base, pl.ds(0, 16)]
                a1 = w[0] * rref[base, pl.ds(16, 16)]
                for q in range(1, NQ):
                    a0 = a0 + w[q] * rref[base + q * 16, pl.ds(0, 16)]
                    a1 = a1 + w[q] * rref[base + q * 16, pl.ds(16, 16)]
                col = zero16 + b
                plsc.store_scatter(oref, [zero16, zero16, lane, col], a0)
                plsc.store_scatter(oref, [zero16, zero16, lane + 16, col], a1)
                return carry

            lax.fori_loop(0, NBW, row_body, 0)

        def stage(c, buf):
            @pl.when(c + 1 < T)
            def _():
                issue(c + 1, (buf + 1) % 2)

            wait_gathers(buf)

            @pl.when(c >= 2)
            def _():
                for src, dst in out_pairs(c, buf):
                    pltpu.make_async_copy(src, dst, osems[buf]).wait()

            compute(buf)
            for src, dst in out_pairs(c, buf):
                pltpu.async_copy(src, dst, osems[buf])

        issue(0, 0)

        def outer(g, carry):
            stage(2 * g, 0)
            stage(2 * g + 1, 1)
            return carry

        lax.fori_loop(0, T // 2, outer, 0)

        for src, dst in out_pairs(T - 2, 0):
            pltpu.make_async_copy(src, dst, osems[0]).wait()
        for src, dst in out_pairs(T - 1, 1):
            pltpu.make_async_copy(src, dst, osems[1]).wait()

    return k


def kernel(tokens, tables, weights):
    flat_table = tables.reshape(NQ * V, D)
    w16 = jnp.broadcast_to(weights.astype(jnp.float32)[:, None], (NQ, 16))
    out4 = _sc_call()(tokens, flat_table, w16)
    out5 = out4.reshape(T, D // 8, 32, 8, NBW)
    return out5.transpose(2, 4, 0, 1, 3).reshape(B, T, D)
